# Initial kernel scaffold; baseline (speedup 1.0000x reference)
import jax
import jax.numpy as jnp
from jax.experimental import pallas as pl


def kernel(x, edge_index, edge_features, params):
    raise NotImplementedError("write your pallas kernel here")


# speedup vs baseline: 2.6963x; 2.6963x over previous
import functools

import jax
import jax.numpy as jnp
from jax import lax
from jax.experimental import pallas as pl
from jax.experimental.pallas import tpu as pltpu
from jax.experimental.pallas import tpu_sc as plsc

_N = 10000
_E = 320000
_L = 128
_OUT = 3

_BE = 2000
_BN = 2000

_NC = 2
_NS = 16
_NW = _NC * _NS
_EPW = _E // _NW
_K = 80
_NCHUNK = _EPW // _K
_NPAD = 10240
_RPT = _NPAD // _NS

_f32 = jnp.float32


def _ln(h, g, b):
    mu = jnp.mean(h, axis=-1, keepdims=True)
    c = h - mu
    var = jnp.mean(c * c, axis=-1, keepdims=True)
    return c * lax.rsqrt(var + 1e-5) * g + b


def _dot(a, b):
    return jnp.dot(a, b, preferred_element_type=_f32)


def _mlp3_body(x_ref, w1, b1, w2, b2, w3, b3, o_ref):
    h = jnp.maximum(_dot(x_ref[...], w1[...]) + b1[...], 0.0)
    h = jnp.maximum(_dot(h, w2[...]) + b2[...], 0.0)
    o_ref[...] = _dot(h, w3[...]) + b3[...]


def _mlp3_ln_body(x_ref, w1, b1, w2, b2, w3, b3, g, bb, o_ref):
    h = jnp.maximum(_dot(x_ref[...], w1[...]) + b1[...], 0.0)
    h = jnp.maximum(_dot(h, w2[...]) + b2[...], 0.0)
    h = _dot(h, w3[...]) + b3[...]
    o_ref[...] = _ln(h, g[...], bb[...])


def _full(a):
    return pl.BlockSpec(a.shape, lambda i: (0, 0))


def _mlp_weights(mlp):
    out = []
    for p in mlp:
        out.append(p["W"])
        out.append(p["b"].reshape(1, -1))
    return out


def _encode(x, mlp, ln, bs):
    rows, din = x.shape
    ws = _mlp_weights(mlp) + [ln["g"].reshape(1, -1), ln["b"].reshape(1, -1)]
    return pl.pallas_call(
        _mlp3_ln_body,
        grid=(rows // bs,),
        in_specs=[pl.BlockSpec((bs, din), lambda i: (i, 0))] + [_full(w) for w in ws],
        out_specs=pl.BlockSpec((bs, _L), lambda i: (i, 0)),
        out_shape=jax.ShapeDtypeStruct((rows, _L), _f32),
    )(x, *ws)


def _decode(nodes, mlp):
    ws = _mlp_weights(mlp)
    return pl.pallas_call(
        _mlp3_body,
        grid=(_N // _BN,),
        in_specs=[pl.BlockSpec((_BN, _L), lambda i: (i, 0))] + [_full(w) for w in ws],
        out_specs=pl.BlockSpec((_BN, _OUT), lambda i: (i, 0)),
        out_shape=jax.ShapeDtypeStruct((_N, _OUT), _f32),
    )(nodes, *ws)


def _project_body(n_ref, ws, wr, os_ref, or_ref):
    nb = n_ref[...]
    os_ref[...] = _dot(nb, ws[...])
    or_ref[...] = _dot(nb, wr[...])


def _project(nodes, w_s, w_r):
    return pl.pallas_call(
        _project_body,
        grid=(_N // _BN,),
        in_specs=[pl.BlockSpec((_BN, _L), lambda i: (i, 0)), _full(w_s), _full(w_r)],
        out_specs=[pl.BlockSpec((_BN, _L), lambda i: (i, 0))] * 2,
        out_shape=[jax.ShapeDtypeStruct((_N, _L), _f32)] * 2,
    )(nodes, w_s, w_r)


def _edge_mlp_body(gs, gr, e_ref, we, b1, w2, b2, w3, b3, g, bb,
                   eupd_ref, enew_ref):
    e = e_ref[...]
    h = jnp.maximum(gs[...] + gr[...] + _dot(e, we[...]) + b1[...], 0.0)
    h = jnp.maximum(_dot(h, w2[...]) + b2[...], 0.0)
    u = _dot(h, w3[...]) + b3[...]
    u = _ln(u, g[...], bb[...])
    eupd_ref[...] = u
    enew_ref[...] = e + u


def _edge_mlp(gs, gr, edges, w_e, mlp, ln):
    ws = [w_e, mlp[0]["b"].reshape(1, -1)] + _mlp_weights(mlp[1:]) + [
        ln["g"].reshape(1, -1), ln["b"].reshape(1, -1)]
    bspec = pl.BlockSpec((_BE, _L), lambda i: (i, 0))
    return pl.pallas_call(
        _edge_mlp_body,
        grid=(_E // _BE,),
        in_specs=[bspec, bspec, bspec] + [_full(w) for w in ws],
        out_specs=[bspec, bspec],
        out_shape=[jax.ShapeDtypeStruct((_E, _L), _f32)] * 2,
    )(gs, gr, edges, *ws)


def _node_mlp_body(n_ref, a0, a1, wn, wa, b1, w2, b2, w3, b3, g, bb, o_ref):
    nodes = n_ref[...]
    agg = a0[...] + a1[...]
    h = jnp.maximum(_dot(nodes, wn[...]) + _dot(agg, wa[...]) + b1[...], 0.0)
    h = jnp.maximum(_dot(h, w2[...]) + b2[...], 0.0)
    u = _dot(h, w3[...]) + b3[...]
    o_ref[...] = nodes + _ln(u, g[...], bb[...])


def _node_mlp(nodes, a0, a1, mlp, ln):
    w1 = mlp[0]["W"]
    ws = [w1[:_L], w1[_L:], mlp[0]["b"].reshape(1, -1)] + _mlp_weights(mlp[1:]) + [
        ln["g"].reshape(1, -1), ln["b"].reshape(1, -1)]
    bspec = pl.BlockSpec((_BN, _L), lambda i: (i, 0))
    return pl.pallas_call(
        _node_mlp_body,
        grid=(_N // _BN,),
        in_specs=[bspec, bspec, bspec] + [_full(w) for w in ws],
        out_specs=bspec,
        out_shape=jax.ShapeDtypeStruct((_N, _L), _f32),
    )(nodes, a0, a1, *ws)


def _sc_gather(ns, nr, senders, receivers):
    mesh = plsc.VectorSubcoreMesh(core_axis_name="c", subcore_axis_name="s")

    @functools.partial(
        pl.kernel,
        out_type=[jax.ShapeDtypeStruct((_E, _L), _f32)] * 2,
        mesh=mesh,
        scratch_types=[
            pltpu.VMEM((_K,), jnp.int32),
            pltpu.VMEM((_K,), jnp.int32),
            pltpu.VMEM((_K, _L), _f32),
            pltpu.VMEM((_K, _L), _f32),
            pltpu.SemaphoreType.DMA,
            pltpu.SemaphoreType.DMA,
        ],
    )
    def k(ns_hbm, nr_hbm, s_hbm, r_hbm, gs_hbm, gr_hbm,
          sidx, ridx, srows, rrows, sem_s, sem_r):
        wid = lax.axis_index("s") * _NC + lax.axis_index("c")

        def body(j, carry):
            base = wid * _EPW + j * _K
            pltpu.sync_copy(s_hbm.at[pl.ds(base, _K)], sidx)
            pltpu.sync_copy(r_hbm.at[pl.ds(base, _K)], ridx)
            cs = pltpu.async_copy(ns_hbm.at[sidx], srows, sem_s)
            cr = pltpu.async_copy(nr_hbm.at[ridx], rrows, sem_r)
            cs.wait()
            cr.wait()
            pltpu.sync_copy(srows, gs_hbm.at[pl.ds(base, _K)])
            pltpu.sync_copy(rrows, gr_hbm.at[pl.ds(base, _K)])
            return carry

        lax.fori_loop(0, _NCHUNK, body, 0)

    return k(ns, nr, senders, receivers)


def _sc_scatter(eupd, receivers, zeros):
    mesh = plsc.VectorSubcoreMesh(core_axis_name="c", subcore_axis_name="s")

    @functools.partial(
        pl.kernel,
        out_type=jax.ShapeDtypeStruct((_NC, _NPAD, _L), _f32),
        mesh=mesh,
        scratch_types=[
            pltpu.VMEM((_K,), jnp.int32),
            pltpu.VMEM((_K, _L), _f32),
            pltpu.VMEM_SHARED((_NPAD, _L), _f32),
        ],
    )
    def k(e_hbm, r_hbm, z_hbm, out_hbm, ridx, rows, acc):
        cid = lax.axis_index("c")
        sid = lax.axis_index("s")
        wid = sid * _NC + cid
        pltpu.sync_copy(z_hbm.at[pl.ds(sid * _RPT, _RPT)],
                        acc.at[pl.ds(sid * _RPT, _RPT)])
        plsc.subcore_barrier()

        def body(j, carry):
            base = wid * _EPW + j * _K
            pltpu.sync_copy(r_hbm.at[pl.ds(base, _K)], ridx)
            pltpu.sync_copy(e_hbm.at[pl.ds(base, _K)], rows)
            pltpu.sync_copy(rows, acc.at[ridx], add=True)
            return carry

        lax.fori_loop(0, _NCHUNK, body, 0)
        plsc.subcore_barrier()
        pltpu.sync_copy(acc.at[pl.ds(sid * _RPT, _RPT)],
                        out_hbm.at[cid, pl.ds(sid * _RPT, _RPT)])

    return k(eupd, receivers, zeros)


def _gnn_step(carry, blk):
    nodes, edges, senders, receivers, zeros = carry
    w1 = blk["edge_mlp"][0]["W"]
    ns, nr = _project(nodes, w1[:_L], w1[_L:2 * _L])
    gs, gr = _sc_gather(ns, nr, senders, receivers)
    eupd, edges_new = _edge_mlp(gs, gr, edges, w1[2 * _L:],
                                blk["edge_mlp"], blk["edge_ln"])
    agg2 = _sc_scatter(eupd, receivers, zeros)
    nodes_new = _node_mlp(nodes, agg2[0, :_N], agg2[1, :_N],
                          blk["node_mlp"], blk["node_ln"])
    return (nodes_new, edges_new, senders, receivers, zeros), None


def kernel(x, edge_index, edge_features, params):
    senders = edge_index[0].astype(jnp.int32)
    receivers = edge_index[1].astype(jnp.int32)
    zeros = jnp.zeros((_NPAD, _L), _f32)

    nodes = _encode(x, params["node_enc"]["mlp"], params["node_enc"]["ln"], _BN)
    edges = _encode(edge_features, params["edge_enc"]["mlp"],
                    params["edge_enc"]["ln"], _BE)

    stacked = jax.tree.map(lambda *xs: jnp.stack(xs), *params["gnn"])
    (nodes, edges, _, _, _), _ = lax.scan(
        _gnn_step, (nodes, edges, senders, receivers, zeros), stacked)

    return _decode(nodes, params["dec"]["mlp"])
